# Initial kernel scaffold; baseline (speedup 1.0000x reference)
import jax
import jax.numpy as jnp
from jax.experimental import pallas as pl


def kernel(x, pos, batch, c1_W1, c1_b1, c1_W2, c1_b2, c2_W1, c2_b1, c2_W2, c2_b2, c3_W1, c3_b1, c3_W2, c3_b2, m_W1, m_b1, m_W2, m_b2, m_W3, m_b3, m_W4, m_b4):
    raise NotImplementedError("write your pallas kernel here")


# speedup vs baseline: 5.2414x; 5.2414x over previous
import functools

import jax
import jax.numpy as jnp
from jax import lax
from jax.experimental import pallas as pl
from jax.experimental.pallas import tpu as pltpu
from jax.experimental.pallas import tpu_sc as plsc

_N = 8192
_K = 20
_BR = 256
_MASKVAL = 1e30
_TAKEN = 2e30

_NW = 32
_CH = 128
_NBUF = 8


def _knn_ab_body(x_ref, batch_ref, w1d_ref, w1b_ref, b1_ref,
                 idx_ref, a_ref, b_ref):
    pid = pl.program_id(0)
    xall = x_ref[...]
    xb = x_ref[pl.ds(pid * _BR, _BR), :]
    sqall = jnp.sum(xall * xall, axis=1)
    sqb = jnp.sum(xb * xb, axis=1)
    dot = lax.dot_general(xb, xall, (((1,), (1,)), ((), ())),
                          preferred_element_type=jnp.float32)
    d2 = sqb[:, None] + sqall[None, :] - 2.0 * dot

    ball = batch_ref[...]
    bb = batch_ref[pl.ds(pid * _BR, _BR)]
    d2 = jnp.where(bb[:, None] != ball[None, :], _MASKVAL, d2)

    iota = lax.broadcasted_iota(jnp.int32, (1, _N), 1)
    cols = []
    for _ in range(_K):
        m = jnp.min(d2, axis=1, keepdims=True)
        cand = jnp.where(d2 <= m, iota, _N)
        a = jnp.min(cand, axis=1, keepdims=True)
        cols.append(a)
        d2 = jnp.where(iota == a, _TAKEN, d2)
    idx_ref[...] = jnp.concatenate(cols, axis=1)

    a_ref[...] = (lax.dot_general(xb, w1d_ref[...], (((1,), (0,)), ((), ())),
                                  preferred_element_type=jnp.float32)
                  + b1_ref[...][None, :])
    b_ref[...] = lax.dot_general(xb, w1b_ref[...], (((1,), (0,)), ((), ())),
                                 preferred_element_type=jnp.float32)


def _knn_ab(xf, batch, w1d, w1b, b1):
    c = xf.shape[1]
    grid = (_N // _BR,)
    return pl.pallas_call(
        _knn_ab_body,
        grid=grid,
        in_specs=[
            pl.BlockSpec((_N, c), lambda i: (0, 0)),
            pl.BlockSpec((_N,), lambda i: (0,)),
            pl.BlockSpec((c, 64), lambda i: (0, 0)),
            pl.BlockSpec((c, 64), lambda i: (0, 0)),
            pl.BlockSpec((64,), lambda i: (0,)),
        ],
        out_specs=[
            pl.BlockSpec((_BR, _K), lambda i: (i, 0)),
            pl.BlockSpec((_BR, 64), lambda i: (i, 0)),
            pl.BlockSpec((_BR, 64), lambda i: (i, 0)),
        ],
        out_shape=[
            jax.ShapeDtypeStruct((_N, _K), jnp.int32),
            jax.ShapeDtypeStruct((_N, 64), jnp.float32),
            jax.ShapeDtypeStruct((_N, 64), jnp.float32),
        ],
    )(xf, batch, w1d, w1b, b1)


def _sc_gather(table, idx3):
    nw, j_chunks, _ = idx3.shape
    mesh = plsc.VectorSubcoreMesh(core_axis_name="c", subcore_axis_name="s")

    @functools.partial(
        pl.kernel,
        out_type=jax.ShapeDtypeStruct((nw * j_chunks, _CH, 64), jnp.float32),
        mesh=mesh,
        scratch_types=[
            pltpu.VMEM((j_chunks, _CH), jnp.int32),
            pltpu.VMEM((_NBUF, _CH, 64), jnp.float32),
            pltpu.SemaphoreType.DMA,
        ],
        compiler_params=pltpu.CompilerParams(use_tc_tiling_on_sc=False),
    )
    def k(table_hbm, idx_hbm, out_hbm, idx_v, rows_v, sem):
        cid = lax.axis_index("c")
        sid = lax.axis_index("s")
        wid = sid * 2 + cid
        pltpu.sync_copy(idx_hbm.at[wid], idx_v)

        def outer(g, _):
            waits = []
            for b in range(_NBUF):
                cp = pltpu.async_copy(
                    table_hbm.at[idx_v.at[g * _NBUF + b]], rows_v.at[b], sem)
                waits.append(cp)
            for cp in waits:
                cp.wait()
            pltpu.sync_copy(
                rows_v,
                out_hbm.at[pl.ds(wid * j_chunks + g * _NBUF, _NBUF)])
            return 0

        lax.fori_loop(0, j_chunks // _NBUF, outer, 0)

    return k(table, idx3)


def _edge_body(g_ref, a_ref, w2_ref, b2_ref, out_ref):
    a = a_ref[...]
    w2 = w2_ref[...]
    acc = None
    for s in range(_K):
        pre = jnp.maximum(g_ref[s] + a, 0.0)
        h = lax.dot_general(pre, w2, (((1,), (0,)), ((), ())),
                            preferred_element_type=jnp.float32)
        acc = h if acc is None else jnp.maximum(acc, h)
    out_ref[...] = acc + b2_ref[...][None, :]


def _edge_max(g, a, w2, b2):
    grid = (_N // _BR,)
    return pl.pallas_call(
        _edge_body,
        grid=grid,
        in_specs=[
            pl.BlockSpec((_K, _BR, 64), lambda i: (0, i, 0)),
            pl.BlockSpec((_BR, 64), lambda i: (i, 0)),
            pl.BlockSpec((64, 64), lambda i: (0, 0)),
            pl.BlockSpec((64,), lambda i: (0,)),
        ],
        out_specs=pl.BlockSpec((_BR, 64), lambda i: (i, 0)),
        out_shape=jax.ShapeDtypeStruct((_N, 64), jnp.float32),
    )(g, a, w2, b2)


def _head_body(x1_ref, x2_ref, x3_ref, wa_ref, wb_ref, wc_ref, b1_ref,
               w2_ref, b2_ref, w3_ref, b3_ref, w4_ref, b4_ref, out_ref):
    mm = lambda x, w: lax.dot_general(x, w, (((1,), (0,)), ((), ())),
                                      preferred_element_type=jnp.float32)
    h = (mm(x1_ref[...], wa_ref[...]) + mm(x2_ref[...], wb_ref[...])
         + mm(x3_ref[...], wc_ref[...]) + b1_ref[...][None, :])
    h = jnp.maximum(h, 0.0)
    h = jnp.maximum(mm(h, w2_ref[...]) + b2_ref[...][None, :], 0.0)
    h = jnp.maximum(mm(h, w3_ref[...]) + b3_ref[...][None, :], 0.0)
    o = mm(h, w4_ref[...]) + b4_ref[...][None, :]
    m = jnp.max(o, axis=1, keepdims=True)
    lse = jnp.log(jnp.sum(jnp.exp(o - m), axis=1, keepdims=True)) + m
    out_ref[...] = o - lse


def _head(x1, x2, x3, wa, wb, wc, b1, w2, b2, w3, b3, w4, b4):
    br = 512
    grid = (_N // br,)
    full = lambda *shape: pl.BlockSpec(shape, lambda i: tuple(0 for _ in shape))
    return pl.pallas_call(
        _head_body,
        grid=grid,
        in_specs=[
            pl.BlockSpec((br, 64), lambda i: (i, 0)),
            pl.BlockSpec((br, 64), lambda i: (i, 0)),
            pl.BlockSpec((br, 64), lambda i: (i, 0)),
            full(64, 1024), full(64, 1024), full(64, 1024), full(1024),
            full(1024, 256), full(256),
            full(256, 128), full(128),
            full(128, 13), full(13),
        ],
        out_specs=pl.BlockSpec((br, 13), lambda i: (i, 0)),
        out_shape=jax.ShapeDtypeStruct((_N, 13), jnp.float32),
    )(x1, x2, x3, wa, wb, wc, b1, w2, b2, w3, b3, w4, b4)


def _edge_conv(xf, batch, w1, b1, w2, b2):
    c = xf.shape[1]
    w1a, w1b = w1[:c], w1[c:]
    idx, a_lin, b_lin = _knn_ab(xf, batch, w1a - w1b, w1b, b1)
    idx3 = idx.T.reshape(_NW, (_N * _K) // (_NW * _CH), _CH)
    rows = _sc_gather(b_lin, idx3)
    g = rows.reshape(_K, _N, 64)
    return _edge_max(g, a_lin, w2, b2)


def kernel(x, pos, batch, c1_W1, c1_b1, c1_W2, c1_b2, c2_W1, c2_b1, c2_W2,
           c2_b2, c3_W1, c3_b1, c3_W2, c3_b2, m_W1, m_b1, m_W2, m_b2, m_W3,
           m_b3, m_W4, m_b4):
    batch = batch.astype(jnp.int32)
    x0 = jnp.concatenate([x, pos], axis=1)
    x1 = _edge_conv(x0, batch, c1_W1, c1_b1, c1_W2, c1_b2)
    x2 = _edge_conv(x1, batch, c2_W1, c2_b1, c2_W2, c2_b2)
    x3 = _edge_conv(x2, batch, c3_W1, c3_b1, c3_W2, c3_b2)
    return _head(x1, x2, x3, m_W1[:64], m_W1[64:128], m_W1[128:], m_b1,
                 m_W2, m_b2, m_W3, m_b3, m_W4, m_b4)


# speedup vs baseline: 10.8808x; 2.0759x over previous
import functools

import jax
import jax.numpy as jnp
from jax import lax
from jax.experimental import pallas as pl
from jax.experimental.pallas import tpu as pltpu
from jax.experimental.pallas import tpu_sc as plsc

_N = 8192
_K = 20
_BR = 256
_MASKVAL = 1e30
_TAKEN = 2e30

_NW = 32
_CH = 128
_NBUF = 8


_CW = 512


def _knn_ab_body(x_ref, batch_ref, w1d_ref, w1b_ref, b1_ref,
                 idx_ref, a_ref, b_ref, d2_scr):
    pid = pl.program_id(0)
    r0 = pid * _BR
    xb = x_ref[pl.ds(r0, _BR), :]
    sqb = jnp.sum(xb * xb, axis=1)
    ball = batch_ref[...]
    bb = batch_ref[pl.ds(r0, _BR)]

    b_lo = jnp.min(bb)
    b_hi = jnp.max(bb)
    c_lo = jnp.sum((ball < b_lo).astype(jnp.int32))
    c_hi = jnp.sum((ball <= b_hi).astype(jnp.int32))
    c0 = (c_lo // _CW) * _CW
    nch = (c_hi - c0 + _CW - 1) // _CW

    def compute_chunk(c, _):
        cs = c0 + c * _CW
        xw = x_ref[pl.ds(cs, _CW), :]
        sqw = jnp.sum(xw * xw, axis=1)
        dotc = lax.dot_general(xb, xw, (((1,), (1,)), ((), ())),
                               preferred_element_type=jnp.float32)
        d2c = sqb[:, None] + sqw[None, :] - 2.0 * dotc
        bw = batch_ref[pl.ds(cs, _CW)]
        d2c = jnp.where(bb[:, None] != bw[None, :], _MASKVAL, d2c)
        d2_scr[:, pl.ds(c * _CW, _CW)] = d2c
        return 0

    lax.fori_loop(0, nch, compute_chunk, 0)

    lidx = lax.broadcasted_iota(jnp.int32, (1, _CW), 1)
    bigi = jnp.int32(_N)
    a_prev = jnp.full((_BR, 1), -1, jnp.int32)
    cols = []
    for _ in range(_K):
        def step(c, carry, a_prev=a_prev):
            m, a = carry
            gi = lidx + (c0 + c * _CW)
            ch = d2_scr[:, pl.ds(c * _CW, _CW)]
            ch = jnp.where(gi == a_prev, _TAKEN, ch)
            d2_scr[:, pl.ds(c * _CW, _CW)] = ch
            cm = jnp.min(ch, axis=1, keepdims=True)
            ca = jnp.min(jnp.where(ch <= cm, gi, bigi), axis=1, keepdims=True)
            a = jnp.where(cm < m, ca,
                          jnp.where(cm == m, jnp.minimum(a, ca), a))
            m = jnp.minimum(m, cm)
            return (m, a)

        m, a = lax.fori_loop(
            0, nch, step,
            (jnp.full((_BR, 1), jnp.inf, jnp.float32),
             jnp.full((_BR, 1), bigi, jnp.int32)))
        cols.append(a)
        a_prev = a
    idx_ref[...] = jnp.concatenate(cols, axis=1)

    a_ref[...] = (lax.dot_general(xb, w1d_ref[...], (((1,), (0,)), ((), ())),
                                  preferred_element_type=jnp.float32)
                  + b1_ref[...][None, :])
    b_ref[...] = lax.dot_general(xb, w1b_ref[...], (((1,), (0,)), ((), ())),
                                 preferred_element_type=jnp.float32)


def _knn_ab(xf, batch, w1d, w1b, b1):
    c = xf.shape[1]
    grid = (_N // _BR,)
    return pl.pallas_call(
        _knn_ab_body,
        grid=grid,
        in_specs=[
            pl.BlockSpec((_N, c), lambda i: (0, 0)),
            pl.BlockSpec((_N,), lambda i: (0,)),
            pl.BlockSpec((c, 64), lambda i: (0, 0)),
            pl.BlockSpec((c, 64), lambda i: (0, 0)),
            pl.BlockSpec((64,), lambda i: (0,)),
        ],
        out_specs=[
            pl.BlockSpec((_BR, _K), lambda i: (i, 0)),
            pl.BlockSpec((_BR, 64), lambda i: (i, 0)),
            pl.BlockSpec((_BR, 64), lambda i: (i, 0)),
        ],
        out_shape=[
            jax.ShapeDtypeStruct((_N, _K), jnp.int32),
            jax.ShapeDtypeStruct((_N, 64), jnp.float32),
            jax.ShapeDtypeStruct((_N, 64), jnp.float32),
        ],
        scratch_shapes=[pltpu.VMEM((_BR, _N), jnp.float32)],
    )(xf, batch, w1d, w1b, b1)


def _sc_gather(table, idx3):
    nw, j_chunks, _ = idx3.shape
    mesh = plsc.VectorSubcoreMesh(core_axis_name="c", subcore_axis_name="s")

    @functools.partial(
        pl.kernel,
        out_type=jax.ShapeDtypeStruct((nw * j_chunks, _CH, 64), jnp.float32),
        mesh=mesh,
        scratch_types=[
            pltpu.VMEM((j_chunks, _CH), jnp.int32),
            pltpu.VMEM((_NBUF, _CH, 64), jnp.float32),
            pltpu.SemaphoreType.DMA,
        ],
        compiler_params=pltpu.CompilerParams(use_tc_tiling_on_sc=False),
    )
    def k(table_hbm, idx_hbm, out_hbm, idx_v, rows_v, sem):
        cid = lax.axis_index("c")
        sid = lax.axis_index("s")
        wid = sid * 2 + cid
        pltpu.sync_copy(idx_hbm.at[wid], idx_v)

        def outer(g, _):
            waits = []
            for b in range(_NBUF):
                cp = pltpu.async_copy(
                    table_hbm.at[idx_v.at[g * _NBUF + b]], rows_v.at[b], sem)
                waits.append(cp)
            for cp in waits:
                cp.wait()
            pltpu.sync_copy(
                rows_v,
                out_hbm.at[pl.ds(wid * j_chunks + g * _NBUF, _NBUF)])
            return 0

        lax.fori_loop(0, j_chunks // _NBUF, outer, 0)

    return k(table, idx3)


def _edge_body(g_ref, a_ref, w2_ref, b2_ref, out_ref):
    a = a_ref[...]
    w2 = w2_ref[...]
    acc = None
    for s in range(_K):
        pre = jnp.maximum(g_ref[s] + a, 0.0)
        h = lax.dot_general(pre, w2, (((1,), (0,)), ((), ())),
                            preferred_element_type=jnp.float32)
        acc = h if acc is None else jnp.maximum(acc, h)
    out_ref[...] = acc + b2_ref[...][None, :]


def _edge_max(g, a, w2, b2):
    grid = (_N // _BR,)
    return pl.pallas_call(
        _edge_body,
        grid=grid,
        in_specs=[
            pl.BlockSpec((_K, _BR, 64), lambda i: (0, i, 0)),
            pl.BlockSpec((_BR, 64), lambda i: (i, 0)),
            pl.BlockSpec((64, 64), lambda i: (0, 0)),
            pl.BlockSpec((64,), lambda i: (0,)),
        ],
        out_specs=pl.BlockSpec((_BR, 64), lambda i: (i, 0)),
        out_shape=jax.ShapeDtypeStruct((_N, 64), jnp.float32),
    )(g, a, w2, b2)


def _head_body(x1_ref, x2_ref, x3_ref, wa_ref, wb_ref, wc_ref, b1_ref,
               w2_ref, b2_ref, w3_ref, b3_ref, w4_ref, b4_ref, out_ref):
    mm = lambda x, w: lax.dot_general(x, w, (((1,), (0,)), ((), ())),
                                      preferred_element_type=jnp.float32)
    h = (mm(x1_ref[...], wa_ref[...]) + mm(x2_ref[...], wb_ref[...])
         + mm(x3_ref[...], wc_ref[...]) + b1_ref[...][None, :])
    h = jnp.maximum(h, 0.0)
    h = jnp.maximum(mm(h, w2_ref[...]) + b2_ref[...][None, :], 0.0)
    h = jnp.maximum(mm(h, w3_ref[...]) + b3_ref[...][None, :], 0.0)
    o = mm(h, w4_ref[...]) + b4_ref[...][None, :]
    m = jnp.max(o, axis=1, keepdims=True)
    lse = jnp.log(jnp.sum(jnp.exp(o - m), axis=1, keepdims=True)) + m
    out_ref[...] = o - lse


def _head(x1, x2, x3, wa, wb, wc, b1, w2, b2, w3, b3, w4, b4):
    br = 512
    grid = (_N // br,)
    full = lambda *shape: pl.BlockSpec(shape, lambda i: tuple(0 for _ in shape))
    return pl.pallas_call(
        _head_body,
        grid=grid,
        in_specs=[
            pl.BlockSpec((br, 64), lambda i: (i, 0)),
            pl.BlockSpec((br, 64), lambda i: (i, 0)),
            pl.BlockSpec((br, 64), lambda i: (i, 0)),
            full(64, 1024), full(64, 1024), full(64, 1024), full(1024),
            full(1024, 256), full(256),
            full(256, 128), full(128),
            full(128, 13), full(13),
        ],
        out_specs=pl.BlockSpec((br, 13), lambda i: (i, 0)),
        out_shape=jax.ShapeDtypeStruct((_N, 13), jnp.float32),
    )(x1, x2, x3, wa, wb, wc, b1, w2, b2, w3, b3, w4, b4)


def _edge_conv(xf, batch, w1, b1, w2, b2):
    c = xf.shape[1]
    w1a, w1b = w1[:c], w1[c:]
    idx, a_lin, b_lin = _knn_ab(xf, batch, w1a - w1b, w1b, b1)
    idx3 = idx.T.reshape(_NW, (_N * _K) // (_NW * _CH), _CH)
    rows = _sc_gather(b_lin, idx3)
    g = rows.reshape(_K, _N, 64)
    return _edge_max(g, a_lin, w2, b2)


def kernel(x, pos, batch, c1_W1, c1_b1, c1_W2, c1_b2, c2_W1, c2_b1, c2_W2,
           c2_b2, c3_W1, c3_b1, c3_W2, c3_b2, m_W1, m_b1, m_W2, m_b2, m_W3,
           m_b3, m_W4, m_b4):
    batch = batch.astype(jnp.int32)
    x0 = jnp.concatenate([x, pos], axis=1)
    x1 = _edge_conv(x0, batch, c1_W1, c1_b1, c1_W2, c1_b2)
    x2 = _edge_conv(x1, batch, c2_W1, c2_b1, c2_W2, c2_b2)
    x3 = _edge_conv(x2, batch, c3_W1, c3_b1, c3_W2, c3_b2)
    return _head(x1, x2, x3, m_W1[:64], m_W1[64:128], m_W1[128:], m_b1,
                 m_W2, m_b2, m_W3, m_b3, m_W4, m_b4)
